# baseline (speedup 1.0000x reference)
import functools

import jax
import jax.numpy as jnp
from jax import lax
from jax.experimental import pallas as pl
from jax.experimental.pallas import tpu as pltpu
from jax.experimental.pallas import tpu_sc as plsc

_LANES = 16


@functools.lru_cache(maxsize=None)
def _build(B, MAXLEN, EMBED):
    info = plsc.get_sparse_core_info()
    NC, NS = info.num_cores, info.num_subcores
    NW = NC * NS
    assert B % NW == 0
    BPW = B // NW
    SUB = 128
    CH = 640
    assert BPW % CH == 0 and CH % SUB == 0
    NCHUNK = BPW // CH
    NSUB = CH // SUB
    EV = EMBED // _LANES

    mesh = plsc.VectorSubcoreMesh(core_axis_name="c", subcore_axis_name="s")

    @functools.partial(
        pl.kernel,
        mesh=mesh,
        compiler_params=pltpu.CompilerParams(use_tc_tiling_on_sc=False),
        out_type=jax.ShapeDtypeStruct((B, EMBED), jnp.float32),
        scratch_types=[
            pltpu.VMEM((BPW,), jnp.int32),
            pltpu.VMEM((MAXLEN, EMBED), jnp.float32),
            pltpu.VMEM((CH, EMBED), jnp.float32),
            pltpu.SemaphoreType.DMA,
        ],
    )
    def embed(x_hbm, tok_hbm, pos_hbm, out_hbm, idx_v, pos_v, buf, sem):
        wid = lax.axis_index("s") * NC + lax.axis_index("c")
        base = wid * BPW
        pltpu.sync_copy(x_hbm.at[pl.ds(base, BPW)], idx_v)
        pltpu.sync_copy(pos_hbm, pos_v)

        for g in range(NCHUNK):
            co = g * CH
            copies = [
                pltpu.async_copy(
                    tok_hbm.at[idx_v.at[pl.ds(co + s * SUB, SUB)]],
                    buf.at[pl.ds(s * SUB, SUB)],
                    sem,
                )
                for s in range(NSUB)
            ]
            for c in copies:
                c.wait()

            row0 = base + co

            def add_row(r, _):
                p = lax.rem(row0 + r, MAXLEN)
                for d in range(EV):
                    sl = pl.ds(d * _LANES, _LANES)
                    buf[r, sl] = buf[r, sl] + pos_v[p, sl]
                return 0

            lax.fori_loop(0, CH, add_row, 0)

            pltpu.sync_copy(buf, out_hbm.at[pl.ds(base + co, CH)])

    return embed


def kernel(x, token_table, pos_table):
    batch, maxlen = x.shape
    embed_dim = token_table.shape[1]
    xf = x.reshape(-1).astype(jnp.int32)
    fn = _build(batch * maxlen, maxlen, embed_dim)
    out = fn(xf, token_table, pos_table)
    return out.reshape(batch, maxlen, embed_dim)


# speedup vs baseline: 1.1368x; 1.1368x over previous
import functools

import jax
import jax.numpy as jnp
from jax import lax
from jax.experimental import pallas as pl
from jax.experimental.pallas import tpu as pltpu
from jax.experimental.pallas import tpu_sc as plsc

_LANES = 16


@functools.lru_cache(maxsize=None)
def _build(B, MAXLEN, EMBED):
    info = plsc.get_sparse_core_info()
    NC, NS = info.num_cores, info.num_subcores
    NW = NC * NS
    assert B % NW == 0
    BPW = B // NW
    assert BPW % MAXLEN == 0
    CH = 400
    assert BPW % CH == 0 and CH % MAXLEN == 0
    NCHUNK = BPW // CH
    POSREP = CH // MAXLEN
    EV = EMBED // _LANES
    SUBS = []
    off = 0
    while off < CH:
        sz = min(128, CH - off)
        SUBS.append((off, sz))
        off += sz

    mesh = plsc.VectorSubcoreMesh(core_axis_name="c", subcore_axis_name="s")

    @functools.partial(
        pl.kernel,
        mesh=mesh,
        compiler_params=pltpu.CompilerParams(use_tc_tiling_on_sc=False),
        out_type=jax.ShapeDtypeStruct((B, EMBED), jnp.float32),
        scratch_types=[
            pltpu.VMEM((BPW,), jnp.int32),
            pltpu.VMEM((CH, EMBED), jnp.float32),
            pltpu.VMEM((2, CH, EMBED), jnp.float32),
            pltpu.SemaphoreType.DMA,
            pltpu.SemaphoreType.DMA,
            pltpu.SemaphoreType.DMA,
            pltpu.SemaphoreType.DMA,
        ],
    )
    def embed(x_hbm, tok_hbm, pos_hbm, out_hbm, idx_v, pos2, bufs, g0, g1, s0, s1):
        gsem = (g0, g1)
        ssem = (s0, s1)
        wid = lax.axis_index("s") * NC + lax.axis_index("c")
        base = wid * BPW
        pltpu.sync_copy(x_hbm.at[pl.ds(base, BPW)], idx_v)
        for rep in range(POSREP):
            pltpu.sync_copy(pos_hbm, pos2.at[pl.ds(rep * MAXLEN, MAXLEN)])

        def issue_gathers(g):
            slot = g % 2
            co = g * CH
            return [
                pltpu.async_copy(
                    tok_hbm.at[idx_v.at[pl.ds(co + off, sz)]],
                    bufs.at[slot].at[pl.ds(off, sz)],
                    gsem[slot],
                )
                for off, sz in SUBS
            ]

        gathers = {0: issue_gathers(0)}
        stores = {}
        for g in range(NCHUNK):
            slot = g % 2
            for c in gathers.pop(g):
                c.wait()
            if g + 1 < NCHUNK:
                nslot = (g + 1) % 2
                if g - 1 in stores:
                    stores.pop(g - 1).wait()
                gathers[g + 1] = issue_gathers(g + 1)

            buf = bufs.at[slot]

            @plsc.parallel_loop(0, CH, 1, unroll=4)
            def add_row(r):
                for d in range(EV):
                    sl = pl.ds(d * _LANES, _LANES)
                    buf[r, sl] = buf[r, sl] + pos2[r, sl]

            stores[g] = pltpu.async_copy(
                buf, out_hbm.at[pl.ds(base + g * CH, CH)], ssem[slot]
            )
        for g in sorted(stores):
            stores.pop(g).wait()

    return embed


def kernel(x, token_table, pos_table):
    batch, maxlen = x.shape
    embed_dim = token_table.shape[1]
    xf = x.reshape(-1).astype(jnp.int32)
    fn = _build(batch * maxlen, maxlen, embed_dim)
    out = fn(xf, token_table, pos_table)
    return out.reshape(batch, maxlen, embed_dim)
